# baseline (speedup 1.0000x reference)
import functools

import jax
import jax.numpy as jnp
from jax import lax
from jax.experimental import pallas as pl
from jax.experimental.pallas import tpu as pltpu
from jax.experimental.pallas import tpu_sc as plsc

N_SRC = 16384
N_QRY = 4096
C_FEAT = 128
K = 3

QB = 128

NC, NS = 2, 16
NW = NC * NS
BPW = N_QRY // NW


CH = 128


def _nn3_body(q_ref, xt_ref, idx_ref, w_ref, d2_ref):
    q = q_ref[...]
    qx, qy, qz = q[:, 0:1], q[:, 1:2], q[:, 2:3]
    inf = jnp.float32(jnp.inf)
    big_f = jnp.float32(N_SRC)

    init = jnp.full((QB, CH), inf, jnp.float32)

    def body(j, carry):
        b1, b2, b3 = carry
        off = j * CH
        xx = xt_ref[0:1, pl.ds(off, CH)]
        yy = xt_ref[1:2, pl.ds(off, CH)]
        zz = xt_ref[2:3, pl.ds(off, CH)]
        dx = qx - xx
        dy = qy - yy
        dz = qz - zz
        v = dx * dx + dy * dy + dz * dz
        d2_ref[:, pl.ds(off, CH)] = v
        t = jnp.maximum(b1, v)
        b1 = jnp.minimum(b1, v)
        u = jnp.maximum(b2, t)
        b2 = jnp.minimum(b2, t)
        b3 = jnp.minimum(b3, u)
        return b1, b2, b3

    b1, b2, b3 = lax.fori_loop(0, N_SRC // CH, body, (init, init, init))

    m1 = jnp.min(b1, axis=1, keepdims=True)
    e1 = b1 == m1
    b1a = jnp.where(e1, b2, b1)
    b2a = jnp.where(e1, b3, b2)
    m2 = jnp.min(b1a, axis=1, keepdims=True)
    e2 = b1a == m2
    b1b = jnp.where(e2, b2a, b1a)
    m3 = jnp.min(b1b, axis=1, keepdims=True)

    iota = lax.broadcasted_iota(jnp.int32, (1, N_SRC), 1).astype(jnp.float32)
    d2 = d2_ref[...]
    i1 = jnp.min(jnp.where(d2 == m1, iota, big_f), axis=1, keepdims=True)
    i2 = jnp.min(jnp.where(d2 == m2, iota, big_f), axis=1, keepdims=True)
    i3 = jnp.min(jnp.where(d2 == m3, iota, big_f), axis=1, keepdims=True)

    d1 = jnp.sqrt(jnp.maximum(m1, 1e-12))
    dd2 = jnp.sqrt(jnp.maximum(m2, 1e-12))
    dd3 = jnp.sqrt(jnp.maximum(m3, 1e-12))
    r1 = 1.0 / (d1 + 1e-8)
    r2 = 1.0 / (dd2 + 1e-8)
    r3 = 1.0 / (dd3 + 1e-8)
    norm = r1 + r2 + r3

    idx_ref[...] = jnp.concatenate([i1, i2, i3], axis=1).astype(jnp.int32).T
    w_ref[...] = jnp.concatenate([r1 / norm, r2 / norm, r3 / norm], axis=1)


def _nn3(new_xyz, xt):
    return pl.pallas_call(
        _nn3_body,
        grid=(N_QRY // QB,),
        in_specs=[
            pl.BlockSpec((QB, 3), lambda i: (i, 0)),
            pl.BlockSpec((3, N_SRC), lambda i: (0, 0)),
        ],
        out_specs=[
            pl.BlockSpec((3, QB), lambda i: (0, i)),
            pl.BlockSpec((QB, 3), lambda i: (i, 0)),
        ],
        out_shape=[
            jax.ShapeDtypeStruct((3, N_QRY), jnp.int32),
            jax.ShapeDtypeStruct((N_QRY, 3), jnp.float32),
        ],
        scratch_shapes=[pltpu.VMEM((QB, N_SRC), jnp.float32)],
    )(new_xyz, xt)


@functools.cache
def _sc_gather_fn():
    mesh = plsc.VectorSubcoreMesh(core_axis_name="c", subcore_axis_name="s")

    @functools.partial(
        pl.kernel,
        mesh=mesh,
        out_type=jax.ShapeDtypeStruct((K, N_QRY, C_FEAT), jnp.float32),
        scratch_types=[
            pltpu.VMEM((K, BPW), jnp.int32),
            pltpu.VMEM((K, BPW, C_FEAT), jnp.float32),
            pltpu.SemaphoreType.DMA,
        ],
    )
    def _sc_gather(idx_hbm, feat_hbm, out_hbm, idx_v, rows_v, sem):
        wid = lax.axis_index("s") * NC + lax.axis_index("c")
        base = wid * BPW
        pltpu.sync_copy(idx_hbm.at[:, pl.ds(base, BPW)], idx_v)
        for k in range(K):
            pltpu.async_copy(feat_hbm.at[idx_v.at[k]], rows_v.at[k], sem)
        for k in range(K):
            pltpu.make_async_copy(feat_hbm.at[idx_v.at[k]], rows_v.at[k],
                                  sem).wait()
        pltpu.sync_copy(rows_v, out_hbm.at[:, pl.ds(base, BPW), :])

    return _sc_gather


def _combine_body(g_ref, w_ref, o_ref):
    w = w_ref[...]
    o_ref[...] = (g_ref[0] * w[:, 0:1]
                  + g_ref[1] * w[:, 1:2]
                  + g_ref[2] * w[:, 2:3])


def _combine(g, w):
    return pl.pallas_call(
        _combine_body,
        grid=(N_QRY // QB,),
        in_specs=[
            pl.BlockSpec((K, QB, C_FEAT), lambda i: (0, i, 0)),
            pl.BlockSpec((QB, 3), lambda i: (i, 0)),
        ],
        out_specs=pl.BlockSpec((QB, C_FEAT), lambda i: (i, 0)),
        out_shape=jax.ShapeDtypeStruct((N_QRY, C_FEAT), jnp.float32),
    )(g, w)


def kernel(xyz, new_xyz, features):
    idx_t, w = _nn3(new_xyz, xyz.T)
    g = _sc_gather_fn()(idx_t, features)
    return _combine(g, w)
